# Initial kernel scaffold; baseline (speedup 1.0000x reference)
import jax
import jax.numpy as jnp
from jax.experimental import pallas as pl


def kernel(sequences, lengths):
    raise NotImplementedError("write your pallas kernel here")


# speedup vs baseline: 14.7368x; 14.7368x over previous
import functools

import jax
import jax.numpy as jnp
from jax import lax
from jax.experimental import pallas as pl
from jax.experimental.pallas import tpu as pltpu

POOL_LVLS = 3
POOL_DIVS = 4
NWIN = sum(POOL_DIVS ** l for l in range(POOL_LVLS))

T = 4096
C = 128
NB8 = T // 8
NB64 = T // 64

NEG_INF = float("-inf")


def _window_bounds(L, level, div_index):
    ndiv = POOL_DIVS ** level
    div_length = (L + (ndiv - 1)) // ndiv
    if ndiv <= 1:
        div_start = jnp.zeros((), jnp.int32)
    else:
        q = ndiv - 1
        r = (L - div_length) * div_index
        quotient = r // q
        rem = r - quotient * q
        div_start = quotient + (2 * rem > q).astype(jnp.int32)
    return div_start, div_length


def _spp_kernel(len_ref, x_ref, out_ref, bm8_ref, bm64_ref):
    b = pl.program_id(0)
    x = x_ref[0]

    bm8 = jnp.max(x.reshape(NB8, 8, C), axis=1)
    bm8_ref[...] = bm8
    bm64_ref[...] = jnp.max(bm8.reshape(NB64, 8, C), axis=1)

    L = len_ref[b]

    row8 = lax.broadcasted_iota(jnp.int32, (8, C), 0)
    j64 = lax.broadcasted_iota(jnp.int32, (NB64, C), 0)

    w = 0
    for level in range(POOL_LVLS):
        for div_index in range(POOL_DIVS ** level):
            s, dl = _window_bounds(L, level, div_index)
            e = s + dl

            kh = s // 8
            kt = (e - 1) // 8
            jh = kh // 8
            jt = kt // 8

            head = x_ref[0, pl.ds(kh * 8, 8), :]
            hpos = kh * 8 + row8
            head = jnp.where((hpos >= s) & (hpos < e), head, NEG_INF)
            tail = x_ref[0, pl.ds(kt * 8, 8), :]
            tpos = kt * 8 + row8
            tail = jnp.where((tpos >= s) & (tpos < e), tail, NEG_INF)
            acc = jnp.maximum(jnp.max(head, axis=0), jnp.max(tail, axis=0))

            mh = bm8_ref[pl.ds(jh * 8, 8), :]
            mk = jh * 8 + row8
            mh = jnp.where((mk > kh) & (mk < kt), mh, NEG_INF)
            mt = bm8_ref[pl.ds(jt * 8, 8), :]
            mk2 = jt * 8 + row8
            mt = jnp.where((mk2 > kh) & (mk2 < kt), mt, NEG_INF)
            acc = jnp.maximum(acc, jnp.max(mh, axis=0))
            acc = jnp.maximum(acc, jnp.max(mt, axis=0))

            inner = jnp.where((j64 > jh) & (j64 < jt), bm64_ref[...], NEG_INF)
            acc = jnp.maximum(acc, jnp.max(inner, axis=0))

            out_ref[0, w, :] = acc
            w += 1


@jax.jit
def kernel(sequences, lengths):
    B = sequences.shape[0]
    return pl.pallas_call(
        _spp_kernel,
        grid=(B,),
        in_specs=[
            pl.BlockSpec(memory_space=pltpu.SMEM),
            pl.BlockSpec((1, T, C), lambda b: (b, 0, 0)),
        ],
        out_specs=pl.BlockSpec((1, NWIN, C), lambda b: (b, 0, 0)),
        out_shape=jax.ShapeDtypeStruct((B, NWIN, C), jnp.float32),
        scratch_shapes=[
            pltpu.VMEM((NB8, C), jnp.float32),
            pltpu.VMEM((NB64, C), jnp.float32),
        ],
    )(lengths, sequences)
